# Initial kernel scaffold; baseline (speedup 1.0000x reference)
import jax
import jax.numpy as jnp
from jax import lax
from jax.experimental import pallas as pl
from jax.experimental.pallas import tpu as pltpu
from jax.experimental.pallas import tpu_sc as plsc

N = 10000
NP = 10240
E = 320000
D = 128
DH = D // 2
K = 3
C = 80
NT = 16
BR = 80
_MESH = plsc.VectorSubcoreMesh(core_axis_name="c", subcore_axis_name="s")


def _zero16():
    return jnp.zeros((16,), jnp.float32)


def _deg_kernel_body(dst_hbm, out0, out1, ones_v, didx_v, zbuf_v, acc_sh):
    c = lax.axis_index("c")
    s = lax.axis_index("s")

    def ini(i, carry):
        ones_v[pl.ds(i * 16, 16)] = jnp.full((16,), 1.0, jnp.float32)
        zbuf_v[pl.ds(i * 16, 16)] = _zero16()
        return carry

    lax.fori_loop(0, C // 16, ini, 0)

    for j in range(NP // NT // C):
        pltpu.sync_copy(zbuf_v, acc_sh.at[pl.ds(s * (NP // NT) + j * C, C)])
    plsc.subcore_barrier()

    base0 = (c * NT + s) * (E // (2 * NT))

    def body(i, carry):
        pltpu.sync_copy(dst_hbm.at[pl.ds(base0 + i * C, C)], didx_v)
        pltpu.sync_copy(ones_v, acc_sh.at[didx_v], add=True)
        return carry

    lax.fori_loop(0, E // (2 * NT) // C, body, 0)
    plsc.subcore_barrier()

    @pl.when(c == 0)
    def _():
        pltpu.sync_copy(acc_sh.at[pl.ds(s * (NP // NT), NP // NT)],
                        out0.at[pl.ds(s * (NP // NT), NP // NT)])

    @pl.when(c == 1)
    def _():
        pltpu.sync_copy(acc_sh.at[pl.ds(s * (NP // NT), NP // NT)],
                        out1.at[pl.ds(s * (NP // NT), NP // NT)])


def _sc_degree(dst):
    f = pl.kernel(
        _deg_kernel_body,
        out_type=(jax.ShapeDtypeStruct((NP,), jnp.float32),
                  jax.ShapeDtypeStruct((NP,), jnp.float32)),
        mesh=_MESH,
        scratch_types=[
            pltpu.VMEM((C,), jnp.float32),
            pltpu.VMEM((C,), jnp.int32),
            pltpu.VMEM((C,), jnp.float32),
            pltpu.VMEM_SHARED((NP,), jnp.float32),
        ],
    )
    return f(dst)


def _hop_kernel_body(src_hbm, dst_hbm, tblL, tblR, outL, outR,
                     idx_v, didx_v, rows_v, sem, acc_sh):
    c = lax.axis_index("c")
    s = lax.axis_index("s")

    def zi(i, carry):
        for j in range(DH // 16):
            rows_v[i, pl.ds(j * 16, 16)] = _zero16()
        return carry

    lax.fori_loop(0, C, zi, 0)
    for j in range(NP // NT // C):
        pltpu.sync_copy(rows_v,
                        acc_sh.at[pl.ds(s * (NP // NT) + j * C, C), :])
    plsc.subcore_barrier()

    nchunk = E // NT // C

    def run(tbl, out):
        def body(i, carry):
            b = s * (E // NT) + i * C
            pltpu.sync_copy(src_hbm.at[pl.ds(b, C)], idx_v)
            pltpu.sync_copy(dst_hbm.at[pl.ds(b, C)], didx_v)
            pltpu.async_copy(tbl.at[idx_v], rows_v, sem).wait()
            pltpu.sync_copy(rows_v, acc_sh.at[didx_v], add=True)
            return carry

        lax.fori_loop(0, nchunk, body, 0)
        plsc.subcore_barrier()
        pltpu.sync_copy(acc_sh.at[pl.ds(s * (NP // NT), NP // NT), :],
                        out.at[pl.ds(s * (NP // NT), NP // NT), :])

    @pl.when(c == 0)
    def _():
        run(tblL, outL)

    @pl.when(c == 1)
    def _():
        run(tblR, outR)


def _sc_hop(src, dst, gL, gR):
    f = pl.kernel(
        _hop_kernel_body,
        out_type=(jax.ShapeDtypeStruct((NP, DH), jnp.float32),
                  jax.ShapeDtypeStruct((NP, DH), jnp.float32)),
        mesh=_MESH,
        scratch_types=[
            pltpu.VMEM((C,), jnp.int32),
            pltpu.VMEM((C,), jnp.int32),
            pltpu.VMEM((C, DH), jnp.float32),
            pltpu.SemaphoreType.DMA,
            pltpu.VMEM_SHARED((NP, DH), jnp.float32),
        ],
    )
    return f(src, dst, gL, gR)


def _scale0_body(x_ref, d0_ref, d1_ref, gL_ref, gR_ref):
    deg = d0_ref[...] + d1_ref[...] + 1.0
    g = x_ref[...] * lax.rsqrt(deg)
    gL_ref[...] = g[:, :DH]
    gR_ref[...] = g[:, DH:]


def _combine_body(pL_ref, pR_ref, gL_ref, gR_ref, d0_ref, d1_ref,
                  oL_ref, oR_ref):
    rdeg = 1.0 / (d0_ref[...] + d1_ref[...] + 1.0)
    oL_ref[...] = (pL_ref[...] + gL_ref[...]) * rdeg
    oR_ref[...] = (pR_ref[...] + gR_ref[...]) * rdeg


def _final_body(pL_ref, pR_ref, gL_ref, gR_ref, d0_ref, d1_ref,
                w_ref, b_ref, out_ref):
    dinv = lax.rsqrt(d0_ref[...] + d1_ref[...] + 1.0)
    h = jnp.concatenate(
        [pL_ref[...] + gL_ref[...], pR_ref[...] + gR_ref[...]], axis=1) * dinv
    o = jnp.dot(h, w_ref[...], preferred_element_type=jnp.float32) + b_ref[...]
    m = jnp.max(o, axis=1, keepdims=True)
    lse = jnp.log(jnp.sum(jnp.exp(o - m), axis=1, keepdims=True)) + m
    out_ref[...] = o - lse


def _row_spec(width):
    return pl.BlockSpec((BR, width), lambda i: (i, 0))


def _tc_scale0(x, d0, d1):
    return pl.pallas_call(
        _scale0_body,
        grid=(N // BR,),
        in_specs=[_row_spec(D), _row_spec(1), _row_spec(1)],
        out_specs=(_row_spec(DH), _row_spec(DH)),
        out_shape=(jax.ShapeDtypeStruct((N, DH), jnp.float32),
                   jax.ShapeDtypeStruct((N, DH), jnp.float32)),
    )(x, d0, d1)


def _tc_combine(pL, pR, gL, gR, d0, d1):
    return pl.pallas_call(
        _combine_body,
        grid=(N // BR,),
        in_specs=[_row_spec(DH)] * 4 + [_row_spec(1)] * 2,
        out_specs=(_row_spec(DH), _row_spec(DH)),
        out_shape=(jax.ShapeDtypeStruct((N, DH), jnp.float32),
                   jax.ShapeDtypeStruct((N, DH), jnp.float32)),
    )(pL, pR, gL, gR, d0, d1)


def _tc_final(pL, pR, gL, gR, d0, d1, W, b):
    return pl.pallas_call(
        _final_body,
        grid=(N // BR,),
        in_specs=[_row_spec(DH)] * 4 + [_row_spec(1)] * 2 + [
            pl.BlockSpec((D, D), lambda i: (0, 0)),
            pl.BlockSpec((1, D), lambda i: (0, 0)),
        ],
        out_specs=_row_spec(D),
        out_shape=jax.ShapeDtypeStruct((N, D), jnp.float32),
    )(pL, pR, gL, gR, d0, d1, W, b)


def kernel(x, edge_index, W, b):
    src = edge_index[0]
    dst = edge_index[1]
    d0p, d1p = _sc_degree(dst)
    d0 = d0p[:N, None]
    d1 = d1p[:N, None]
    gL, gR = _tc_scale0(x, d0, d1)
    for k in range(K):
        pL, pR = _sc_hop(src, dst, gL, gR)
        pL = pL[:N]
        pR = pR[:N]
        if k < K - 1:
            gL, gR = _tc_combine(pL, pR, gL, gR, d0, d1)
    return _tc_final(pL, pR, gL, gR, d0, d1, W, b.reshape(1, D))


# speedup vs baseline: 9.1669x; 9.1669x over previous
import jax
import jax.numpy as jnp
from jax import lax
from jax.experimental import pallas as pl
from jax.experimental.pallas import tpu as pltpu
from jax.experimental.pallas import tpu_sc as plsc

N = 10000
NP = 10240
E = 320000
D = 128
K = 3
C = 80
NT = 16
NW = 32
BR = 80
_MESH = plsc.VectorSubcoreMesh(core_axis_name="c", subcore_axis_name="s")


def _zero16():
    return jnp.zeros((16,), jnp.float32)


def _deg_kernel_body(dst_hbm, out_hbm, didx_v, dacc_v):
    c = lax.axis_index("c")
    s = lax.axis_index("s")
    wid = c * NT + s
    ones16 = jnp.full((16,), 1.0, jnp.float32)

    def zi(i, carry):
        dacc_v[pl.ds(i * 16, 16)] = _zero16()
        return carry

    lax.fori_loop(0, NP // 16, zi, 0)

    base0 = wid * (E // NW)

    def body(i, carry):
        pltpu.sync_copy(dst_hbm.at[pl.ds(base0 + i * C, C)], didx_v)
        for j in range(C // 16):
            idx = didx_v[pl.ds(j * 16, 16)]
            plsc.addupdate_scatter(dacc_v, [idx], ones16)
        return carry

    lax.fori_loop(0, E // NW // C, body, 0)
    pltpu.sync_copy(dacc_v, out_hbm.at[pl.ds(wid * NP, NP)])


def _sc_degree(dst):
    f = pl.kernel(
        _deg_kernel_body,
        out_type=jax.ShapeDtypeStruct((NW * NP,), jnp.float32),
        mesh=_MESH,
        scratch_types=[
            pltpu.VMEM((C,), jnp.int32),
            pltpu.VMEM((NP,), jnp.float32),
        ],
        compiler_params=pltpu.CompilerParams(needs_layout_passes=False),
    )
    return f(dst)


def _hop_kernel_body(src_hbm, dst_hbm, tbl, out0, out1,
                     idx_v, didx_v, rows_v, sem, acc_sh):
    c = lax.axis_index("c")
    s = lax.axis_index("s")

    def zi(i, carry):
        for j in range(D // 16):
            rows_v[i, pl.ds(j * 16, 16)] = _zero16()
        return carry

    lax.fori_loop(0, C, zi, 0)
    for j in range(NP // NT // C):
        pltpu.sync_copy(rows_v,
                        acc_sh.at[pl.ds(s * (NP // NT) + j * C, C), :])
    plsc.subcore_barrier()

    base0 = (c * NT + s) * (E // NW)

    def body(i, carry):
        b = base0 + i * C
        pltpu.sync_copy(src_hbm.at[pl.ds(b, C)], idx_v)
        pltpu.sync_copy(dst_hbm.at[pl.ds(b, C)], didx_v)
        pltpu.async_copy(tbl.at[idx_v], rows_v, sem).wait()
        pltpu.sync_copy(rows_v, acc_sh.at[didx_v], add=True)
        return carry

    lax.fori_loop(0, E // NW // C, body, 0)
    plsc.subcore_barrier()

    row0 = s * (NP // NT)

    @pl.when(c == 0)
    def _():
        pltpu.sync_copy(acc_sh.at[pl.ds(row0, NP // NT), :],
                        out0.at[pl.ds(row0, NP // NT), :])

    @pl.when(c == 1)
    def _():
        pltpu.sync_copy(acc_sh.at[pl.ds(row0, NP // NT), :],
                        out1.at[pl.ds(row0, NP // NT), :])


def _sc_hop(src, dst, g):
    f = pl.kernel(
        _hop_kernel_body,
        out_type=(jax.ShapeDtypeStruct((NP, D), jnp.float32),
                  jax.ShapeDtypeStruct((NP, D), jnp.float32)),
        mesh=_MESH,
        scratch_types=[
            pltpu.VMEM((C,), jnp.int32),
            pltpu.VMEM((C,), jnp.int32),
            pltpu.VMEM((C, D), jnp.float32),
            pltpu.SemaphoreType.DMA,
            pltpu.VMEM_SHARED((NP, D), jnp.float32),
        ],
    )
    return f(src, dst, g)


def _scale0_body(x_ref, dT_ref, g_ref):
    deg = jnp.sum(dT_ref[...], axis=1, keepdims=True) + 1.0
    g_ref[...] = x_ref[...] * lax.rsqrt(deg)


def _combine_body(p0_ref, p1_ref, g_ref, dT_ref, o_ref):
    deg = jnp.sum(dT_ref[...], axis=1, keepdims=True) + 1.0
    o_ref[...] = (p0_ref[...] + p1_ref[...] + g_ref[...]) / deg


def _final_body(p0_ref, p1_ref, g_ref, dT_ref, w_ref, b_ref, out_ref):
    deg = jnp.sum(dT_ref[...], axis=1, keepdims=True) + 1.0
    h = (p0_ref[...] + p1_ref[...] + g_ref[...]) * lax.rsqrt(deg)
    o = jnp.dot(h, w_ref[...], preferred_element_type=jnp.float32) + b_ref[...]
    m = jnp.max(o, axis=1, keepdims=True)
    lse = jnp.log(jnp.sum(jnp.exp(o - m), axis=1, keepdims=True)) + m
    out_ref[...] = o - lse


def _row_spec(width):
    return pl.BlockSpec((BR, width), lambda i: (i, 0))


def _tc_scale0(x, dT):
    return pl.pallas_call(
        _scale0_body,
        grid=(N // BR,),
        in_specs=[_row_spec(D), _row_spec(NW)],
        out_specs=_row_spec(D),
        out_shape=jax.ShapeDtypeStruct((N, D), jnp.float32),
    )(x, dT)


def _tc_combine(p0, p1, g, dT):
    return pl.pallas_call(
        _combine_body,
        grid=(N // BR,),
        in_specs=[_row_spec(D)] * 3 + [_row_spec(NW)],
        out_specs=_row_spec(D),
        out_shape=jax.ShapeDtypeStruct((N, D), jnp.float32),
    )(p0, p1, g, dT)


def _tc_final(p0, p1, g, dT, W, b):
    return pl.pallas_call(
        _final_body,
        grid=(N // BR,),
        in_specs=[_row_spec(D)] * 3 + [_row_spec(NW)] + [
            pl.BlockSpec((D, D), lambda i: (0, 0)),
            pl.BlockSpec((1, D), lambda i: (0, 0)),
        ],
        out_specs=_row_spec(D),
        out_shape=jax.ShapeDtypeStruct((N, D), jnp.float32),
    )(p0, p1, g, dT, W, b)


def kernel(x, edge_index, W, b):
    src = edge_index[0]
    dst = edge_index[1]
    dall = _sc_degree(dst)
    dT = dall.reshape(NW, NP)[:, :N].T
    g = _tc_scale0(x, dT)
    for k in range(K):
        p0, p1 = _sc_hop(src, dst, g)
        p0 = p0[:N]
        p1 = p1[:N]
        if k < K - 1:
            g = _tc_combine(p0, p1, g, dT)
    return _tc_final(p0, p1, g, dT, W, b.reshape(1, D))
